# Initial kernel scaffold; baseline (speedup 1.0000x reference)
import jax
import jax.numpy as jnp
from jax.experimental import pallas as pl


def kernel(inputs, eu_gmf, ei_gmf, eu_mlp, ei_mlp, W1, b1, W2, b2, Wp, bp):
    raise NotImplementedError("write your pallas kernel here")


# speedup vs baseline: 6.2868x; 6.2868x over previous
import functools

import jax
import jax.numpy as jnp
from jax import lax
from jax.experimental import pallas as pl
from jax.experimental.pallas import tpu as pltpu
from jax.experimental.pallas import tpu_sc as plsc

U = 100000
S = 100000
D = 128
DM = 2 * D
B = 16384

NC = 2
NS = 16
NW = NC * NS
BPW = B // NW
CH = 128
NCH = BPW // CH


def _sc_gather(inputs, eu_gmf, ei_gmf, eu_mlp, ei_mlp):
  mesh = plsc.VectorSubcoreMesh(core_axis_name="c", subcore_axis_name="s")

  @functools.partial(
      pl.kernel,
      out_type=(
          jax.ShapeDtypeStruct((B, DM), jnp.float32),
          jax.ShapeDtypeStruct((B, DM), jnp.float32),
          jax.ShapeDtypeStruct((B, D), jnp.float32),
          jax.ShapeDtypeStruct((B, D), jnp.float32),
      ),
      mesh=mesh,
      scratch_types=[
          pltpu.VMEM((NCH, CH), jnp.int32),
          pltpu.VMEM((NCH, CH), jnp.int32),
          pltpu.VMEM((CH, DM), jnp.float32),
          pltpu.VMEM((CH, D), jnp.float32),
          pltpu.SemaphoreType.DMA,
      ],
  )
  def body(idx_hbm, eu_gmf_h, ei_gmf_h, eu_mlp_h, ei_mlp_h,
           um_out, im_out, ug_out, ig_out,
           idx_u, idx_s, buf_w, buf_n, sem):
    wid = lax.axis_index("s") * NC + lax.axis_index("c")
    base = wid * BPW
    for ci in range(NCH):
      pltpu.sync_copy(idx_hbm.at[0, pl.ds(base + ci * CH, CH)], idx_u.at[ci])
      pltpu.sync_copy(idx_hbm.at[1, pl.ds(base + ci * CH, CH)], idx_s.at[ci])
    for tbl, idx, out, buf in (
        (eu_mlp_h, idx_u, um_out, buf_w),
        (ei_mlp_h, idx_s, im_out, buf_w),
        (eu_gmf_h, idx_u, ug_out, buf_n),
        (ei_gmf_h, idx_s, ig_out, buf_n),
    ):
      for ci in range(NCH):
        pltpu.async_copy(tbl.at[idx.at[ci]], buf, sem).wait()
        pltpu.sync_copy(buf, out.at[pl.ds(base + ci * CH, CH)])

  return body(inputs, eu_gmf, ei_gmf, eu_mlp, ei_mlp)


def _dense_body(um_ref, im_ref, ug_ref, ig_ref,
                w1u_ref, w1i_ref, b1_ref, w2_ref, b2_ref, wp_ref, bp_ref,
                emb_ref, y_ref):
  h = jnp.dot(um_ref[...], w1u_ref[...], preferred_element_type=jnp.float32)
  h += jnp.dot(im_ref[...], w1i_ref[...], preferred_element_type=jnp.float32)
  h = jnp.maximum(h + b1_ref[...], 0.0)
  h2 = jnp.dot(h, w2_ref[...], preferred_element_type=jnp.float32)
  h2 = jnp.maximum(h2 + b2_ref[...], 0.0)
  gmf = ug_ref[...] * ig_ref[...]
  emb = jnp.concatenate([gmf, h2], axis=-1)
  emb_ref[...] = emb
  y_ref[...] = jnp.sum(emb * wp_ref[...], axis=-1) + bp_ref[0]


def _tc_dense(um, im, ug, ig, W1, b1, W2, b2, Wp, bp):
  bs = 1024
  grid = (B // bs,)
  w1u = W1.T[:DM]
  w1i = W1.T[DM:]
  w2 = W2.T
  b1r = b1.reshape(1, -1)
  b2r = b2.reshape(1, -1)
  wpr = Wp.reshape(1, -1)
  bpr = bp.reshape(1)
  return pl.pallas_call(
      _dense_body,
      grid=grid,
      in_specs=[
          pl.BlockSpec((bs, DM), lambda i: (i, 0)),
          pl.BlockSpec((bs, DM), lambda i: (i, 0)),
          pl.BlockSpec((bs, D), lambda i: (i, 0)),
          pl.BlockSpec((bs, D), lambda i: (i, 0)),
          pl.BlockSpec((DM, DM), lambda i: (0, 0)),
          pl.BlockSpec((DM, DM), lambda i: (0, 0)),
          pl.BlockSpec((1, DM), lambda i: (0, 0)),
          pl.BlockSpec((DM, D), lambda i: (0, 0)),
          pl.BlockSpec((1, D), lambda i: (0, 0)),
          pl.BlockSpec((1, DM), lambda i: (0, 0)),
          pl.BlockSpec(memory_space=pltpu.SMEM),
      ],
      out_specs=[
          pl.BlockSpec((bs, DM), lambda i: (i, 0)),
          pl.BlockSpec((bs,), lambda i: (i,)),
      ],
      out_shape=[
          jax.ShapeDtypeStruct((B, DM), jnp.float32),
          jax.ShapeDtypeStruct((B,), jnp.float32),
      ],
  )(um, im, ug, ig, w1u, w1i, b1r, w2, b2r, wpr, bpr)


def kernel(inputs, eu_gmf, ei_gmf, eu_mlp, ei_mlp, W1, b1, W2, b2, Wp, bp):
  um, im, ug, ig = _sc_gather(inputs, eu_gmf, ei_gmf, eu_mlp, ei_mlp)
  embeds, y = _tc_dense(um, im, ug, ig, W1, b1, W2, b2, Wp, bp)
  return embeds, y


# speedup vs baseline: 7.1558x; 1.1382x over previous
import functools

import jax
import jax.numpy as jnp
from jax import lax
from jax.experimental import pallas as pl
from jax.experimental.pallas import tpu as pltpu
from jax.experimental.pallas import tpu_sc as plsc

U = 100000
S = 100000
D = 128
DM = 2 * D
B = 16384

NC = 2
NS = 16
NW = NC * NS
BPW = B // NW
CH = 128
NCH = BPW // CH


def _sc_gather(inputs, eu_gmf, ei_gmf, eu_mlp, ei_mlp):
  mesh = plsc.VectorSubcoreMesh(core_axis_name="c", subcore_axis_name="s")

  @functools.partial(
      pl.kernel,
      out_type=(
          jax.ShapeDtypeStruct((B, DM), jnp.float32),
          jax.ShapeDtypeStruct((B, DM), jnp.float32),
          jax.ShapeDtypeStruct((B, D), jnp.float32),
          jax.ShapeDtypeStruct((B, D), jnp.float32),
      ),
      mesh=mesh,
      scratch_types=[
          pltpu.VMEM((BPW,), jnp.int32),
          pltpu.VMEM((BPW,), jnp.int32),
          pltpu.VMEM((CH, DM), jnp.float32),
          pltpu.VMEM((CH, DM), jnp.float32),
          pltpu.VMEM((CH, D), jnp.float32),
          pltpu.VMEM((CH, D), jnp.float32),
          pltpu.SemaphoreType.DMA,
          pltpu.SemaphoreType.DMA,
      ],
  )
  def body(idx_hbm, eu_gmf_h, ei_gmf_h, eu_mlp_h, ei_mlp_h,
           um_out, im_out, ug_out, ig_out,
           idx_u, idx_s, bw0, bw1, bn0, bn1, sem0, sem1):
    wid = lax.axis_index("s") * NC + lax.axis_index("c")
    base = wid * BPW
    pltpu.sync_copy(idx_hbm.at[0, pl.ds(base, BPW)], idx_u)
    pltpu.sync_copy(idx_hbm.at[1, pl.ds(base, BPW)], idx_s)
    sems = (sem0, sem1)

    def run_ring(tables, bufs):
      tasks = [(tbl, idx, out, ci)
               for tbl, idx, out in tables for ci in range(NCH)]
      descs = [None, None]

      def start(t):
        tbl, idx, out, ci = tasks[t]
        slot = t % 2
        descs[slot] = pltpu.async_copy(
            tbl.at[idx.at[pl.ds(ci * CH, CH)]], bufs[slot], sems[slot])

      start(0)
      for t in range(len(tasks)):
        slot = t % 2
        if t + 1 < len(tasks):
          start(t + 1)
        descs[slot].wait()
        _, _, out, ci = tasks[t]
        pltpu.sync_copy(bufs[slot], out.at[pl.ds(base + ci * CH, CH)])

    run_ring(((eu_mlp_h, idx_u, um_out), (ei_mlp_h, idx_s, im_out)),
             (bw0, bw1))
    run_ring(((eu_gmf_h, idx_u, ug_out), (ei_gmf_h, idx_s, ig_out)),
             (bn0, bn1))

  return body(inputs, eu_gmf, ei_gmf, eu_mlp, ei_mlp)


def _dense_body(um_ref, im_ref, ug_ref, ig_ref,
                w1u_ref, w1i_ref, b1_ref, w2_ref, b2_ref, wp_ref, bp_ref,
                emb_ref, y_ref):
  h = jnp.dot(um_ref[...], w1u_ref[...], preferred_element_type=jnp.float32)
  h += jnp.dot(im_ref[...], w1i_ref[...], preferred_element_type=jnp.float32)
  h = jnp.maximum(h + b1_ref[...], 0.0)
  h2 = jnp.dot(h, w2_ref[...], preferred_element_type=jnp.float32)
  h2 = jnp.maximum(h2 + b2_ref[...], 0.0)
  gmf = ug_ref[...] * ig_ref[...]
  emb = jnp.concatenate([gmf, h2], axis=-1)
  emb_ref[...] = emb
  y_ref[...] = jnp.sum(emb * wp_ref[...], axis=-1) + bp_ref[0]


def _tc_dense(um, im, ug, ig, W1, b1, W2, b2, Wp, bp):
  bs = 1024
  grid = (B // bs,)
  w1u = W1.T[:DM]
  w1i = W1.T[DM:]
  w2 = W2.T
  b1r = b1.reshape(1, -1)
  b2r = b2.reshape(1, -1)
  wpr = Wp.reshape(1, -1)
  bpr = bp.reshape(1)
  return pl.pallas_call(
      _dense_body,
      grid=grid,
      in_specs=[
          pl.BlockSpec((bs, DM), lambda i: (i, 0)),
          pl.BlockSpec((bs, DM), lambda i: (i, 0)),
          pl.BlockSpec((bs, D), lambda i: (i, 0)),
          pl.BlockSpec((bs, D), lambda i: (i, 0)),
          pl.BlockSpec((DM, DM), lambda i: (0, 0)),
          pl.BlockSpec((DM, DM), lambda i: (0, 0)),
          pl.BlockSpec((1, DM), lambda i: (0, 0)),
          pl.BlockSpec((DM, D), lambda i: (0, 0)),
          pl.BlockSpec((1, D), lambda i: (0, 0)),
          pl.BlockSpec((1, DM), lambda i: (0, 0)),
          pl.BlockSpec(memory_space=pltpu.SMEM),
      ],
      out_specs=[
          pl.BlockSpec((bs, DM), lambda i: (i, 0)),
          pl.BlockSpec((bs,), lambda i: (i,)),
      ],
      out_shape=[
          jax.ShapeDtypeStruct((B, DM), jnp.float32),
          jax.ShapeDtypeStruct((B,), jnp.float32),
      ],
  )(um, im, ug, ig, w1u, w1i, b1r, w2, b2r, wpr, bpr)


def kernel(inputs, eu_gmf, ei_gmf, eu_mlp, ei_mlp, W1, b1, W2, b2, Wp, bp):
  um, im, ug, ig = _sc_gather(inputs, eu_gmf, ei_gmf, eu_mlp, ei_mlp)
  embeds, y = _tc_dense(um, im, ug, ig, W1, b1, W2, b2, Wp, bp)
  return embeds, y


# speedup vs baseline: 7.3329x; 1.0248x over previous
import functools

import jax
import jax.numpy as jnp
from jax import lax
from jax.experimental import pallas as pl
from jax.experimental.pallas import tpu as pltpu
from jax.experimental.pallas import tpu_sc as plsc

U = 100000
S = 100000
D = 128
DM = 2 * D
B = 16384

NC = 2
NS = 16
NW = NC * NS
BPW = B // NW
CH = 128
NCH = BPW // CH


def _sc_gather(inputs, eu_gmf, ei_gmf, eu_mlp, ei_mlp):
  mesh = plsc.VectorSubcoreMesh(core_axis_name="c", subcore_axis_name="s")

  @functools.partial(
      pl.kernel,
      out_type=(
          jax.ShapeDtypeStruct((B, DM), jnp.float32),
          jax.ShapeDtypeStruct((B, DM), jnp.float32),
          jax.ShapeDtypeStruct((B, D), jnp.float32),
          jax.ShapeDtypeStruct((B, D), jnp.float32),
      ),
      mesh=mesh,
      scratch_types=[
          pltpu.VMEM((BPW,), jnp.int32),
          pltpu.VMEM((BPW,), jnp.int32),
          pltpu.VMEM((CH, DM), jnp.float32),
          pltpu.VMEM((CH, DM), jnp.float32),
          pltpu.VMEM((CH, D), jnp.float32),
          pltpu.VMEM((CH, D), jnp.float32),
          pltpu.SemaphoreType.DMA,
          pltpu.SemaphoreType.DMA,
      ],
  )
  def body(idx_hbm, eu_gmf_h, ei_gmf_h, eu_mlp_h, ei_mlp_h,
           um_out, im_out, ug_out, ig_out,
           idx_u, idx_s, bw0, bw1, bn0, bn1, sem0, sem1):
    wid = lax.axis_index("s") * NC + lax.axis_index("c")
    base = wid * BPW
    pltpu.sync_copy(idx_hbm.at[0, pl.ds(base, BPW)], idx_u)
    pltpu.sync_copy(idx_hbm.at[1, pl.ds(base, BPW)], idx_s)
    sems = (sem0, sem1)

    def run_ring(tables, bufs):
      tasks = [(tbl, idx, out, ci)
               for tbl, idx, out in tables for ci in range(NCH)]
      descs = [None, None]

      def start(t):
        tbl, idx, out, ci = tasks[t]
        slot = t % 2
        descs[slot] = pltpu.async_copy(
            tbl.at[idx.at[pl.ds(ci * CH, CH)]], bufs[slot], sems[slot])

      start(0)
      for t in range(len(tasks)):
        slot = t % 2
        if t + 1 < len(tasks):
          start(t + 1)
        descs[slot].wait()
        _, _, out, ci = tasks[t]
        pltpu.sync_copy(bufs[slot], out.at[pl.ds(base + ci * CH, CH)])

    run_ring(((eu_mlp_h, idx_u, um_out), (ei_mlp_h, idx_s, im_out)),
             (bw0, bw1))
    run_ring(((eu_gmf_h, idx_u, ug_out), (ei_gmf_h, idx_s, ig_out)),
             (bn0, bn1))

  return body(inputs, eu_gmf, ei_gmf, eu_mlp, ei_mlp)


def _dense_body(um_ref, im_ref, ug_ref, ig_ref,
                w1u_ref, w1i_ref, b1_ref, w2_ref, b2_ref, wp_ref, bp_ref,
                emb_ref, y_ref):
  bf = jnp.bfloat16
  h = jnp.dot(um_ref[...].astype(bf), w1u_ref[...].astype(bf),
              preferred_element_type=jnp.float32)
  h += jnp.dot(im_ref[...].astype(bf), w1i_ref[...].astype(bf),
               preferred_element_type=jnp.float32)
  h = jnp.maximum(h + b1_ref[...], 0.0)
  h2 = jnp.dot(h.astype(bf), w2_ref[...].astype(bf),
               preferred_element_type=jnp.float32)
  h2 = jnp.maximum(h2 + b2_ref[...], 0.0)
  gmf = ug_ref[...] * ig_ref[...]
  emb = jnp.concatenate([gmf, h2], axis=-1)
  emb_ref[...] = emb
  y_ref[...] = jnp.dot(emb, wp_ref[...],
                       preferred_element_type=jnp.float32) + bp_ref[0]


def _tc_dense(um, im, ug, ig, W1, b1, W2, b2, Wp, bp):
  bs = 2048
  grid = (B // bs,)
  w1u = W1.T[:DM]
  w1i = W1.T[DM:]
  w2 = W2.T
  b1r = b1.reshape(1, -1)
  b2r = b2.reshape(1, -1)
  wpr = Wp.reshape(-1, 1)
  bpr = bp.reshape(1)
  embeds, y2 = pl.pallas_call(
      _dense_body,
      grid=grid,
      in_specs=[
          pl.BlockSpec((bs, DM), lambda i: (i, 0)),
          pl.BlockSpec((bs, DM), lambda i: (i, 0)),
          pl.BlockSpec((bs, D), lambda i: (i, 0)),
          pl.BlockSpec((bs, D), lambda i: (i, 0)),
          pl.BlockSpec((DM, DM), lambda i: (0, 0)),
          pl.BlockSpec((DM, DM), lambda i: (0, 0)),
          pl.BlockSpec((1, DM), lambda i: (0, 0)),
          pl.BlockSpec((DM, D), lambda i: (0, 0)),
          pl.BlockSpec((1, D), lambda i: (0, 0)),
          pl.BlockSpec((DM, 1), lambda i: (0, 0)),
          pl.BlockSpec(memory_space=pltpu.SMEM),
      ],
      out_specs=[
          pl.BlockSpec((bs, DM), lambda i: (i, 0)),
          pl.BlockSpec((bs, 1), lambda i: (i, 0)),
      ],
      out_shape=[
          jax.ShapeDtypeStruct((B, DM), jnp.float32),
          jax.ShapeDtypeStruct((B, 1), jnp.float32),
      ],
  )(um, im, ug, ig, w1u, w1i, b1r, w2, b2r, wpr, bpr)
  return embeds, y2.reshape(-1)


def kernel(inputs, eu_gmf, ei_gmf, eu_mlp, ei_mlp, W1, b1, W2, b2, Wp, bp):
  um, im, ug, ig = _sc_gather(inputs, eu_gmf, ei_gmf, eu_mlp, ei_mlp)
  embeds, y = _tc_dense(um, im, ug, ig, W1, b1, W2, b2, Wp, bp)
  return embeds, y
